# Initial kernel scaffold; baseline (speedup 1.0000x reference)
import jax
import jax.numpy as jnp
from jax.experimental import pallas as pl


def kernel(x, g_attn, Wq, Wkv, Wc, g_ff, Wr, sw1, sw2, scp, ew1, ew2, ecp, start_posn, use_kv_cache):
    raise NotImplementedError("write your pallas kernel here")


# speedup vs baseline: 1.6091x; 1.6091x over previous
import functools
import math

import jax
import jax.numpy as jnp
from jax.experimental import pallas as pl
from jax.experimental.pallas import tpu as pltpu

B = 1
S = 2048
D = 1024
H = 16
KVH = 8
DK = D // H
HID = 1024
E = 8
TOPK = 2
NSH = 1
THETA = 10000.0
EPS = 1e-6

BS = 256
QB = 512
LANES = 128


def _rms(x, g):
    return g * (x / jnp.sqrt(jnp.mean(x * x, axis=-1, keepdims=True) + EPS))


def _k1_body(x_ref, g_ref, wq_ref, wkv_ref, cos_ref, sin_ref, q_ref, k_ref, v_ref):
    xn = _rms(x_ref[...], g_ref[...])
    q = jnp.dot(xn, wq_ref[...], preferred_element_type=jnp.float32)
    kv = jnp.dot(xn, wkv_ref[...], preferred_element_type=jnp.float32)
    k = kv[:, : KVH * DK]
    v = kv[:, KVH * DK :]
    cos = cos_ref[...]
    sin = sin_ref[...]

    def rope(t, c, s):
        even = jax.lax.broadcasted_iota(jnp.int32, t.shape, 1) % 2 == 0
        n = t.shape[1]
        swap = jnp.where(even, pltpu.roll(t, n - 1, 1), pltpu.roll(t, 1, 1))
        return t * c + swap * s

    qr = rope(q, cos, sin) * (1.0 / math.sqrt(DK))
    kr = rope(k, cos[:, : KVH * DK], sin[:, : KVH * DK])
    for hh in range(H):
        q_ref[hh] = qr[:, hh * DK : (hh + 1) * DK]
    for hh in range(KVH):
        k_ref[hh] = kr[:, hh * DK : (hh + 1) * DK]
        v_ref[hh] = v[:, hh * DK : (hh + 1) * DK]


def _k1(x2, g_attn, Wq, Wkv, cosD, sinD):
    return pl.pallas_call(
        _k1_body,
        grid=(S // BS,),
        in_specs=[
            pl.BlockSpec((BS, D), lambda i: (i, 0)),
            pl.BlockSpec((D,), lambda i: (0,)),
            pl.BlockSpec((D, D), lambda i: (0, 0)),
            pl.BlockSpec((D, D), lambda i: (0, 0)),
            pl.BlockSpec((BS, D), lambda i: (i, 0)),
            pl.BlockSpec((BS, D), lambda i: (i, 0)),
        ],
        out_specs=[
            pl.BlockSpec((H, BS, DK), lambda i: (0, i, 0)),
            pl.BlockSpec((KVH, BS, DK), lambda i: (0, i, 0)),
            pl.BlockSpec((KVH, BS, DK), lambda i: (0, i, 0)),
        ],
        out_shape=[
            jax.ShapeDtypeStruct((H, S, DK), jnp.float32),
            jax.ShapeDtypeStruct((KVH, S, DK), jnp.float32),
            jax.ShapeDtypeStruct((KVH, S, DK), jnp.float32),
        ],
        compiler_params=pltpu.CompilerParams(
            dimension_semantics=("arbitrary",)),
    )(x2, g_attn, Wq, Wkv, cosD, sinD)


def _k2_body(q_ref, k_ref, v_ref, o_ref):
    sb = pl.program_id(1)
    q = q_ref[0]
    k = k_ref[0]
    v = v_ref[0]
    logits = jax.lax.dot_general(q, k, (((1,), (1,)), ((), ())),
                                 preferred_element_type=jnp.float32)
    row = jax.lax.broadcasted_iota(jnp.int32, logits.shape, 0) + sb * QB
    col = jax.lax.broadcasted_iota(jnp.int32, logits.shape, 1)
    logits = jnp.where(col <= row, logits, -1e30)
    m = jnp.max(logits, axis=-1, keepdims=True)
    p = jnp.exp(logits - m)
    p = p / jnp.sum(p, axis=-1, keepdims=True)
    o_ref[0] = jnp.dot(p, v, preferred_element_type=jnp.float32)


def _k2(q, k, v):
    return pl.pallas_call(
        _k2_body,
        grid=(H, S // QB),
        in_specs=[
            pl.BlockSpec((1, QB, DK), lambda h, sb: (h, sb, 0)),
            pl.BlockSpec((1, S, DK), lambda h, sb: (h // (H // KVH), 0, 0)),
            pl.BlockSpec((1, S, DK), lambda h, sb: (h // (H // KVH), 0, 0)),
        ],
        out_specs=pl.BlockSpec((1, QB, DK), lambda h, sb: (h, sb, 0)),
        out_shape=jax.ShapeDtypeStruct((H, S, DK), jnp.float32),
        compiler_params=pltpu.CompilerParams(
            dimension_semantics=("arbitrary", "arbitrary")),
    )(q, k, v)


def _k3_body(attn_ref, x_ref, wc_ref, g_ref, wr_ref, h_ref, hn_ref, pr_ref, cw_ref):
    attn = jnp.concatenate([attn_ref[hh] for hh in range(H)], axis=-1)
    h = x_ref[...] + jnp.dot(attn, wc_ref[...],
                             preferred_element_type=jnp.float32)
    h_ref[...] = h
    hn = _rms(h, g_ref[...])
    hn_ref[...] = hn
    rl = jnp.dot(hn, wr_ref[...], preferred_element_type=jnp.float32)
    lane = jax.lax.broadcasted_iota(jnp.int32, rl.shape, 1)
    valid = lane < E
    rl = jnp.where(valid, rl, -1e30)
    m = jnp.max(rl, axis=-1, keepdims=True)
    p = jnp.exp(rl - m)
    p = p / jnp.sum(p, axis=-1, keepdims=True)
    pr_ref[...] = p
    m1 = jnp.max(p, axis=-1, keepdims=True)
    i1 = jnp.min(jnp.where(p == m1, lane, E), axis=-1, keepdims=True)
    p2 = jnp.where(valid & (lane != i1), p, -1.0)
    m2 = jnp.max(p2, axis=-1, keepdims=True)
    i2 = jnp.min(jnp.where(p2 == m2, lane, E), axis=-1, keepdims=True)
    tot = m1 + m2
    cw = jnp.where(lane == i1, m1 / tot, 0.0) + jnp.where(lane == i2, m2 / tot, 0.0)
    cw_ref[...] = cw


def _k3(attn, x2, Wc, g_ff, Wr_pad):
    return pl.pallas_call(
        _k3_body,
        grid=(S // BS,),
        in_specs=[
            pl.BlockSpec((H, BS, DK), lambda i: (0, i, 0)),
            pl.BlockSpec((BS, D), lambda i: (i, 0)),
            pl.BlockSpec((D, D), lambda i: (0, 0)),
            pl.BlockSpec((D,), lambda i: (0,)),
            pl.BlockSpec((D, LANES), lambda i: (0, 0)),
        ],
        out_specs=[
            pl.BlockSpec((BS, D), lambda i: (i, 0)),
            pl.BlockSpec((BS, D), lambda i: (i, 0)),
            pl.BlockSpec((BS, LANES), lambda i: (i, 0)),
            pl.BlockSpec((BS, LANES), lambda i: (i, 0)),
        ],
        out_shape=[
            jax.ShapeDtypeStruct((S, D), jnp.float32),
            jax.ShapeDtypeStruct((S, D), jnp.float32),
            jax.ShapeDtypeStruct((S, LANES), jnp.float32),
            jax.ShapeDtypeStruct((S, LANES), jnp.float32),
        ],
        compiler_params=pltpu.CompilerParams(
            dimension_semantics=("arbitrary",)),
    )(attn, x2, Wc, g_ff, Wr_pad)


def _k5_body(hn_ref, h_ref, w1_ref, w2_ref, cp_ref, o_ref):
    hn = hn_ref[...]
    a1 = jnp.dot(hn, w1_ref[...], preferred_element_type=jnp.float32)
    a2 = jnp.dot(hn, w2_ref[...], preferred_element_type=jnp.float32)
    act = jax.nn.silu(a1) * a2
    o_ref[...] = h_ref[...] + jnp.dot(act, cp_ref[...],
                                      preferred_element_type=jnp.float32)


def _k5(hn, h, w1, w2, cp):
    return pl.pallas_call(
        _k5_body,
        grid=(S // BS,),
        in_specs=[
            pl.BlockSpec((BS, D), lambda i: (i, 0)),
            pl.BlockSpec((BS, D), lambda i: (i, 0)),
            pl.BlockSpec((D, HID), lambda i: (0, 0)),
            pl.BlockSpec((D, HID), lambda i: (0, 0)),
            pl.BlockSpec((HID, D), lambda i: (0, 0)),
        ],
        out_specs=pl.BlockSpec((BS, D), lambda i: (i, 0)),
        out_shape=jax.ShapeDtypeStruct((S, D), jnp.float32),
        compiler_params=pltpu.CompilerParams(
            dimension_semantics=("arbitrary",)),
    )(hn, h, w1, w2, cp)


def _k4_body(hn_ref, base_ref, cw_ref, w1_ref, w2_ref, cp_ref, o_ref):
    e = pl.program_id(0)
    sb = pl.program_id(1)
    rows = pl.ds(sb * BS, BS)
    hn = hn_ref[rows, :]
    cwb = cw_ref[rows, :]
    lane = jax.lax.broadcasted_iota(jnp.int32, cwb.shape, 1)
    cwcol = jnp.sum(jnp.where(lane == e, cwb, 0.0), axis=-1, keepdims=True)
    a1 = jnp.dot(hn, w1_ref[0], preferred_element_type=jnp.float32)
    a2 = jnp.dot(hn, w2_ref[0], preferred_element_type=jnp.float32)
    act = jax.nn.silu(a1) * a2
    contrib = cwcol * jnp.dot(act, cp_ref[0], preferred_element_type=jnp.float32)

    @pl.when(e == 0)
    def _():
        o_ref[rows, :] = base_ref[rows, :] + contrib

    @pl.when(e > 0)
    def _():
        o_ref[rows, :] = o_ref[rows, :] + contrib


def _k4(hn, base, cw_pad, ew1, ew2, ecp):
    full = lambda e, sb: (0, 0)
    return pl.pallas_call(
        _k4_body,
        grid=(E, S // BS),
        in_specs=[
            pl.BlockSpec((S, D), full),
            pl.BlockSpec((S, D), full),
            pl.BlockSpec((S, LANES), full),
            pl.BlockSpec((1, D, HID), lambda e, sb: (e, 0, 0)),
            pl.BlockSpec((1, D, HID), lambda e, sb: (e, 0, 0)),
            pl.BlockSpec((1, HID, D), lambda e, sb: (e, 0, 0)),
        ],
        out_specs=pl.BlockSpec((S, D), full),
        out_shape=jax.ShapeDtypeStruct((S, D), jnp.float32),
        compiler_params=pltpu.CompilerParams(
            dimension_semantics=("arbitrary", "arbitrary")),
    )(hn, base, cw_pad, ew1, ew2, ecp)


def kernel(x, g_attn, Wq, Wkv, Wc, g_ff, Wr, sw1, sw2, scp, ew1, ew2, ecp,
           start_posn=0, use_kv_cache=False):
    x2 = x.reshape(S, D)
    pairs = jnp.arange(DK // 2, dtype=jnp.float32)
    freqs = 1.0 / THETA ** (2.0 * pairs / DK)
    pos = jnp.arange(S, dtype=jnp.float32) + jnp.asarray(start_posn, jnp.float32)
    ang = pos[:, None] * freqs[None, :]
    cos2 = jnp.repeat(jnp.cos(ang), 2, axis=1)
    sgn = jnp.tile(jnp.array([-1.0, 1.0], jnp.float32), DK // 2)
    sin2 = jnp.repeat(jnp.sin(ang), 2, axis=1) * sgn[None, :]
    cosD = jnp.tile(cos2, (1, H))
    sinD = jnp.tile(sin2, (1, H))

    Wr_pad = jnp.zeros((D, LANES), jnp.float32).at[:, :E].set(Wr)

    q, k, v = _k1(x2, g_attn, Wq, Wkv, cosD, sinD)
    attn = _k2(q, k, v)
    h, hn, probs_pad, cw_pad = _k3(attn, x2, Wc, g_ff, Wr_pad)
    base = _k5(hn, h, sw1[0], sw2[0], scp[0])
    out = _k4(hn, base, cw_pad, ew1, ew2, ecp)
    return out.reshape(B, S, D), probs_pad[:, :E].reshape(B, S, E)
